# baseline (speedup 1.0000x reference)
import jax
import jax.numpy as jnp
from jax import lax
from jax.experimental import pallas as pl
from jax.experimental.pallas import tpu as pltpu

N_WIN = 8
TOPK = 4
NUM_HEADS = 12
TAU = 2.0
V_TH = 1.0
THRESH = TAU * V_TH


W_PER_STEP = 2


def _qkv_g_kernel(x_ref, w_ref, b_ref, q_ref, g_ref, r_ref):
    win = x_ref.shape[2]
    C = x_ref.shape[3]
    hd = C // NUM_HEADS
    x2 = x_ref[0].reshape(W_PER_STEP * win, C)
    qkv = jnp.dot(x2, w_ref[...], preferred_element_type=jnp.float32) + b_ref[...]
    s = qkv >= THRESH
    q_ref[0] = s[:, :C].astype(jnp.int8).reshape(W_PER_STEP, win, C)
    sf = s[:, C:].astype(jnp.bfloat16)
    for wi in range(W_PER_STEP):
        r0 = wi * win
        r_ref[0, wi] = jnp.sum(x_ref[0, wi], axis=0, keepdims=True)
        for h in range(NUM_HEADS):
            kh = sf[r0:r0 + win, h * hd:(h + 1) * hd]
            vh = sf[r0:r0 + win, C + h * hd:C + (h + 1) * hd]
            g_ref[0, :, wi, 0, h * hd:(h + 1) * hd] = lax.dot_general(
                kh, vh, (((0,), (0,)), ((), ())),
                preferred_element_type=jnp.float32).astype(jnp.bfloat16)


def _routing_mask(r):
    a = lax.dot_general(r, r, (((1,), (1,)), ((), ())),
                        preferred_element_type=jnp.float32)
    col = lax.broadcasted_iota(jnp.int32, (N_WIN, N_WIN), 1)
    rank = jnp.zeros((N_WIN, N_WIN), jnp.float32)
    for jp in range(N_WIN):
        aj = a[:, jp:jp + 1]
        rank = rank + (aj > a).astype(jnp.float32)
        rank = rank + ((aj == a) & (col > jp)).astype(jnp.float32)
    return (rank < float(TOPK)).astype(jnp.bfloat16)


def _attn_proj_kernel(r_ref, q_ref, g_ref, wp_ref, bp_ref, o_ref, kv_scr, o_scr):
    wp = pl.program_id(1)
    C = wp_ref.shape[0]
    hd = C // NUM_HEADS

    @pl.when(wp == 0)
    def _():
        mask = _routing_mask(jnp.sum(r_ref[:, :, 0, :], axis=0))
        for d in range(hd):
            g_d = g_ref[0, d, :, 0, :]
            kv_scr[:, d, :] = lax.dot_general(
                mask, g_d, (((1,), (0,)), ((), ())),
                preferred_element_type=jnp.float32)

    @pl.when(wp > 0)
    def _():
        w = wp - 1
        q = q_ref[0, 0].astype(jnp.float32)
        kv = kv_scr[w]
        for h in range(NUM_HEADS):
            qh = q[:, h * hd:(h + 1) * hd]
            kvh = kv[:, h * hd:(h + 1) * hd]
            o_scr[:, h * hd:(h + 1) * hd] = jnp.dot(
                qh, kvh, preferred_element_type=jnp.float32)
        y = jnp.dot(o_scr[...], wp_ref[...],
                    preferred_element_type=jnp.float32) + bp_ref[...]
        o_ref[0, 0] = (y >= THRESH).astype(jnp.float32)


def kernel(x, W_qkv, b_qkv, W_proj, b_proj):
    T, B, L, C = x.shape
    assert B == 1
    n_win = N_WIN
    win = L // n_win
    hd = C // NUM_HEADS
    x4 = x.reshape(T, n_win, win, C)
    b2_qkv = b_qkv.reshape(1, 3 * C)
    b2_proj = b_proj.reshape(1, C)

    ws = W_PER_STEP
    q, g, region = pl.pallas_call(
        _qkv_g_kernel,
        grid=(n_win // ws, T),
        in_specs=[
            pl.BlockSpec((1, ws, win, C), lambda w, t: (t, w, 0, 0)),
            pl.BlockSpec((C, 3 * C), lambda w, t: (0, 0)),
            pl.BlockSpec((1, 3 * C), lambda w, t: (0, 0)),
        ],
        out_specs=[
            pl.BlockSpec((1, ws, win, C), lambda w, t: (t, w, 0, 0)),
            pl.BlockSpec((1, hd, ws, 1, C), lambda w, t: (t, 0, w, 0, 0)),
            pl.BlockSpec((1, ws, 1, C), lambda w, t: (t, w, 0, 0)),
        ],
        out_shape=[
            jax.ShapeDtypeStruct((T, n_win, win, C), jnp.int8),
            jax.ShapeDtypeStruct((T, hd, n_win, 1, C), jnp.bfloat16),
            jax.ShapeDtypeStruct((T, n_win, 1, C), jnp.float32),
        ],
        compiler_params=pltpu.CompilerParams(
            dimension_semantics=("parallel", "parallel")),
    )(x4, W_qkv, b2_qkv)

    out = pl.pallas_call(
        _attn_proj_kernel,
        grid=(T, n_win + 1),
        in_specs=[
            pl.BlockSpec((T, n_win, 1, C), lambda t, wp: (0, 0, 0, 0)),
            pl.BlockSpec((1, 1, win, C),
                         lambda t, wp: (t, jnp.maximum(wp - 1, 0), 0, 0)),
            pl.BlockSpec((1, hd, n_win, 1, C), lambda t, wp: (t, 0, 0, 0, 0)),
            pl.BlockSpec((C, C), lambda t, wp: (0, 0)),
            pl.BlockSpec((1, C), lambda t, wp: (0, 0)),
        ],
        out_specs=pl.BlockSpec((1, 1, win, C),
                               lambda t, wp: (t, jnp.maximum(wp - 1, 0), 0, 0)),
        out_shape=jax.ShapeDtypeStruct((T, n_win, win, C), jnp.float32),
        scratch_shapes=[pltpu.VMEM((n_win, hd, C), jnp.float32),
                        pltpu.VMEM((win, C), jnp.float32)],
        compiler_params=pltpu.CompilerParams(
            dimension_semantics=("parallel", "arbitrary")),
    )(region, q, g, W_proj, b2_proj)

    return out.reshape(T, B, L, C)


# speedup vs baseline: 1.1190x; 1.1190x over previous
import jax
import jax.numpy as jnp
from jax import lax
from jax.experimental import pallas as pl
from jax.experimental.pallas import tpu as pltpu

N_WIN = 8
TOPK = 4
NUM_HEADS = 12
TAU = 2.0
V_TH = 1.0
THRESH = TAU * V_TH


W_PER_STEP = 4
W_PER_STEP_ATTN = 2


def _qkv_g_kernel(x_ref, w_ref, b_ref, q_ref, g_ref, r_ref):
    win = x_ref.shape[2]
    C = x_ref.shape[3]
    hd = C // NUM_HEADS
    x2 = x_ref[0].reshape(W_PER_STEP * win, C)
    qkv = jnp.dot(x2, w_ref[...], preferred_element_type=jnp.float32) + b_ref[...]
    s = qkv >= THRESH
    q_ref[0] = s[:, :C].astype(jnp.int8).reshape(W_PER_STEP, win, C)
    sf = s[:, C:].astype(jnp.bfloat16)
    for wi in range(W_PER_STEP):
        r0 = wi * win
        r_ref[0, wi] = jnp.sum(x_ref[0, wi], axis=0, keepdims=True)
        for h in range(NUM_HEADS):
            kh = sf[r0:r0 + win, h * hd:(h + 1) * hd]
            vh = sf[r0:r0 + win, C + h * hd:C + (h + 1) * hd]
            g_ref[0, :, wi, 0, h * hd:(h + 1) * hd] = lax.dot_general(
                kh, vh, (((0,), (0,)), ((), ())),
                preferred_element_type=jnp.float32).astype(jnp.bfloat16)


def _routing_mask(r):
    a = lax.dot_general(r, r, (((1,), (1,)), ((), ())),
                        preferred_element_type=jnp.float32)
    col = lax.broadcasted_iota(jnp.int32, (N_WIN, N_WIN), 1)
    rank = jnp.zeros((N_WIN, N_WIN), jnp.float32)
    for jp in range(N_WIN):
        aj = a[:, jp:jp + 1]
        rank = rank + (aj > a).astype(jnp.float32)
        rank = rank + ((aj == a) & (col > jp)).astype(jnp.float32)
    return (rank < float(TOPK)).astype(jnp.bfloat16)


def _attn_proj_kernel(r_ref, q_ref, g_ref, wp_ref, bp_ref, o_ref, kv_scr, o_scr):
    wp = pl.program_id(1)
    C = wp_ref.shape[0]
    hd = C // NUM_HEADS

    @pl.when(wp == 0)
    def _():
        mask = _routing_mask(jnp.sum(r_ref[:, :, 0, :], axis=0))
        for d in range(hd):
            g_d = g_ref[0, d, :, 0, :]
            kv_scr[:, d, :] = lax.dot_general(
                mask, g_d, (((1,), (0,)), ((), ())),
                preferred_element_type=jnp.float32)

    @pl.when(wp > 0)
    def _():
        win = q_ref.shape[2]
        wa = W_PER_STEP_ATTN
        for wi in range(wa):
            q = q_ref[0, wi].astype(jnp.float32)
            kv = kv_scr[wa * (wp - 1) + wi]
            for h in range(NUM_HEADS):
                qh = q[:, h * hd:(h + 1) * hd]
                kvh = kv[:, h * hd:(h + 1) * hd]
                o_scr[wi * win:(wi + 1) * win, h * hd:(h + 1) * hd] = jnp.dot(
                    qh, kvh, preferred_element_type=jnp.float32)
        y = jnp.dot(o_scr[...], wp_ref[...],
                    preferred_element_type=jnp.float32) + bp_ref[...]
        o_ref[0] = ((y >= THRESH).astype(jnp.float32)
                    .reshape(wa, win, y.shape[1]))


def kernel(x, W_qkv, b_qkv, W_proj, b_proj):
    T, B, L, C = x.shape
    assert B == 1
    n_win = N_WIN
    win = L // n_win
    hd = C // NUM_HEADS
    x4 = x.reshape(T, n_win, win, C)
    b2_qkv = b_qkv.reshape(1, 3 * C)
    b2_proj = b_proj.reshape(1, C)

    ws = W_PER_STEP
    q, g, region = pl.pallas_call(
        _qkv_g_kernel,
        grid=(n_win // ws, T),
        in_specs=[
            pl.BlockSpec((1, ws, win, C), lambda w, t: (t, w, 0, 0)),
            pl.BlockSpec((C, 3 * C), lambda w, t: (0, 0)),
            pl.BlockSpec((1, 3 * C), lambda w, t: (0, 0)),
        ],
        out_specs=[
            pl.BlockSpec((1, ws, win, C), lambda w, t: (t, w, 0, 0)),
            pl.BlockSpec((1, hd, ws, 1, C), lambda w, t: (t, 0, w, 0, 0)),
            pl.BlockSpec((1, ws, 1, C), lambda w, t: (t, w, 0, 0)),
        ],
        out_shape=[
            jax.ShapeDtypeStruct((T, n_win, win, C), jnp.int8),
            jax.ShapeDtypeStruct((T, hd, n_win, 1, C), jnp.bfloat16),
            jax.ShapeDtypeStruct((T, n_win, 1, C), jnp.float32),
        ],
        compiler_params=pltpu.CompilerParams(
            dimension_semantics=("parallel", "parallel")),
    )(x4, W_qkv, b2_qkv)

    wa = W_PER_STEP_ATTN
    out = pl.pallas_call(
        _attn_proj_kernel,
        grid=(T, n_win // wa + 1),
        in_specs=[
            pl.BlockSpec((T, n_win, 1, C), lambda t, wp: (0, 0, 0, 0)),
            pl.BlockSpec((1, wa, win, C),
                         lambda t, wp: (t, jnp.maximum(wp - 1, 0), 0, 0)),
            pl.BlockSpec((1, hd, n_win, 1, C), lambda t, wp: (t, 0, 0, 0, 0)),
            pl.BlockSpec((C, C), lambda t, wp: (0, 0)),
            pl.BlockSpec((1, C), lambda t, wp: (0, 0)),
        ],
        out_specs=pl.BlockSpec((1, wa, win, C),
                               lambda t, wp: (t, jnp.maximum(wp - 1, 0), 0, 0)),
        out_shape=jax.ShapeDtypeStruct((T, n_win, win, C), jnp.float32),
        scratch_shapes=[pltpu.VMEM((n_win, hd, C), jnp.float32),
                        pltpu.VMEM((wa * win, C), jnp.float32)],
        compiler_params=pltpu.CompilerParams(
            dimension_semantics=("parallel", "arbitrary")),
    )(region, q, g, W_proj, b2_proj)

    return out.reshape(T, B, L, C)


# speedup vs baseline: 1.1256x; 1.0059x over previous
import jax
import jax.numpy as jnp
from jax import lax
from jax.experimental import pallas as pl
from jax.experimental.pallas import tpu as pltpu

N_WIN = 8
TOPK = 4
NUM_HEADS = 12
TAU = 2.0
V_TH = 1.0
THRESH = TAU * V_TH


W_PER_STEP = 8
W_PER_STEP_ATTN = 4


def _qkv_g_kernel(x_ref, w_ref, b_ref, q_ref, g_ref, r_ref):
    win = x_ref.shape[2]
    C = x_ref.shape[3]
    hd = C // NUM_HEADS
    x2 = x_ref[0].reshape(W_PER_STEP * win, C)
    qkv = jnp.dot(x2, w_ref[...], preferred_element_type=jnp.float32) + b_ref[...]
    s = qkv >= THRESH
    q_ref[0] = s[:, :C].astype(jnp.int8).reshape(W_PER_STEP, win, C)
    sf = s[:, C:].astype(jnp.bfloat16)
    for wi in range(W_PER_STEP):
        r0 = wi * win
        r_ref[0, wi] = jnp.sum(x_ref[0, wi], axis=0, keepdims=True)
        for h in range(NUM_HEADS):
            kh = sf[r0:r0 + win, h * hd:(h + 1) * hd]
            vh = sf[r0:r0 + win, C + h * hd:C + (h + 1) * hd]
            g_ref[0, :, wi, 0, h * hd:(h + 1) * hd] = lax.dot_general(
                kh, vh, (((0,), (0,)), ((), ())),
                preferred_element_type=jnp.float32).astype(jnp.bfloat16)


def _routing_mask(r):
    a = lax.dot_general(r, r, (((1,), (1,)), ((), ())),
                        preferred_element_type=jnp.float32)
    col = lax.broadcasted_iota(jnp.int32, (N_WIN, N_WIN), 1)
    rank = jnp.zeros((N_WIN, N_WIN), jnp.float32)
    for jp in range(N_WIN):
        aj = a[:, jp:jp + 1]
        rank = rank + (aj > a).astype(jnp.float32)
        rank = rank + ((aj == a) & (col > jp)).astype(jnp.float32)
    return (rank < float(TOPK)).astype(jnp.bfloat16)


def _attn_proj_kernel(r_ref, q_ref, g_ref, wp_ref, bp_ref, o_ref, kv_scr, o_scr):
    wp = pl.program_id(1)
    C = wp_ref.shape[0]
    hd = C // NUM_HEADS

    @pl.when(wp == 0)
    def _():
        mask = _routing_mask(jnp.sum(r_ref[:, :, 0, :], axis=0))
        for d in range(hd):
            g_d = g_ref[0, d, :, 0, :]
            kv_scr[:, d, :] = lax.dot_general(
                mask, g_d, (((1,), (0,)), ((), ())),
                preferred_element_type=jnp.float32)

    @pl.when(wp > 0)
    def _():
        win = q_ref.shape[2]
        wa = W_PER_STEP_ATTN
        for wi in range(wa):
            q = q_ref[0, wi].astype(jnp.float32)
            kv = kv_scr[wa * (wp - 1) + wi]
            for h in range(NUM_HEADS):
                qh = q[:, h * hd:(h + 1) * hd]
                kvh = kv[:, h * hd:(h + 1) * hd]
                o_scr[wi * win:(wi + 1) * win, h * hd:(h + 1) * hd] = jnp.dot(
                    qh, kvh, preferred_element_type=jnp.float32)
        y = jnp.dot(o_scr[...], wp_ref[...],
                    preferred_element_type=jnp.float32) + bp_ref[...]
        o_ref[0] = ((y >= THRESH).astype(jnp.float32)
                    .reshape(wa, win, y.shape[1]))


def kernel(x, W_qkv, b_qkv, W_proj, b_proj):
    T, B, L, C = x.shape
    assert B == 1
    n_win = N_WIN
    win = L // n_win
    hd = C // NUM_HEADS
    x4 = x.reshape(T, n_win, win, C)
    b2_qkv = b_qkv.reshape(1, 3 * C)
    b2_proj = b_proj.reshape(1, C)

    ws = W_PER_STEP
    q, g, region = pl.pallas_call(
        _qkv_g_kernel,
        grid=(n_win // ws, T),
        in_specs=[
            pl.BlockSpec((1, ws, win, C), lambda w, t: (t, w, 0, 0)),
            pl.BlockSpec((C, 3 * C), lambda w, t: (0, 0)),
            pl.BlockSpec((1, 3 * C), lambda w, t: (0, 0)),
        ],
        out_specs=[
            pl.BlockSpec((1, ws, win, C), lambda w, t: (t, w, 0, 0)),
            pl.BlockSpec((1, hd, ws, 1, C), lambda w, t: (t, 0, w, 0, 0)),
            pl.BlockSpec((1, ws, 1, C), lambda w, t: (t, w, 0, 0)),
        ],
        out_shape=[
            jax.ShapeDtypeStruct((T, n_win, win, C), jnp.int8),
            jax.ShapeDtypeStruct((T, hd, n_win, 1, C), jnp.bfloat16),
            jax.ShapeDtypeStruct((T, n_win, 1, C), jnp.float32),
        ],
        compiler_params=pltpu.CompilerParams(
            dimension_semantics=("parallel", "parallel")),
    )(x4, W_qkv, b2_qkv)

    wa = W_PER_STEP_ATTN
    out = pl.pallas_call(
        _attn_proj_kernel,
        grid=(T, n_win // wa + 1),
        in_specs=[
            pl.BlockSpec((T, n_win, 1, C), lambda t, wp: (0, 0, 0, 0)),
            pl.BlockSpec((1, wa, win, C),
                         lambda t, wp: (t, jnp.maximum(wp - 1, 0), 0, 0)),
            pl.BlockSpec((1, hd, n_win, 1, C), lambda t, wp: (t, 0, 0, 0, 0)),
            pl.BlockSpec((C, C), lambda t, wp: (0, 0)),
            pl.BlockSpec((1, C), lambda t, wp: (0, 0)),
        ],
        out_specs=pl.BlockSpec((1, wa, win, C),
                               lambda t, wp: (t, jnp.maximum(wp - 1, 0), 0, 0)),
        out_shape=jax.ShapeDtypeStruct((T, n_win, win, C), jnp.float32),
        scratch_shapes=[pltpu.VMEM((n_win, hd, C), jnp.float32),
                        pltpu.VMEM((wa * win, C), jnp.float32)],
        compiler_params=pltpu.CompilerParams(
            dimension_semantics=("parallel", "arbitrary")),
    )(region, q, g, W_proj, b2_proj)

    return out.reshape(T, B, L, C)
